# baseline (speedup 1.0000x reference)
import functools

import jax
import jax.numpy as jnp
from jax import lax
from jax.experimental import pallas as pl
from jax.experimental.pallas import tpu as pltpu
from jax.experimental.pallas import tpu_sc as plsc

MEM_DIM = 32
TIME_DIM = 64

_TWO_OVER_PI = 0.6366197723675814
_PIO2_HI = 1.5707963267341256
_PIO2_LO = 6.077100506506192e-11


def _fast_cos(x):
    q = jnp.round(x * _TWO_OVER_PI)
    r = x - q * _PIO2_HI
    r = r - q * _PIO2_LO
    qi = q.astype(jnp.int32)
    r2 = r * r
    c = 1.0 + r2 * (-0.5 + r2 * (4.16666667e-2 + r2 * -1.38888889e-3))
    s = r * (1.0 + r2 * (-1.66666667e-1 + r2 * (8.33333333e-3
                                                + r2 * -1.98412698e-4)))
    odd = (qi & 1) == 1
    neg = (qi & 2) == 2
    val = jnp.where(odd, s, c)
    return jnp.where(odd ^ neg, -val, val)


def _tc_proj(tableT, wcol, *, cb):
    n = tableT.shape[1]
    g = pl.cdiv(n, cb)
    rows_per_blk = cb // 128

    def body(w_ref, t_ref, o_ref):
        s = jnp.sum(t_ref[...] * w_ref[...], axis=0)
        o_ref[...] = s.reshape(rows_per_blk, 128)

    return pl.pallas_call(
        body,
        grid=(g,),
        in_specs=[
            pl.BlockSpec((MEM_DIM, 1), lambda i: (0, 0)),
            pl.BlockSpec((MEM_DIM, cb), lambda i: (0, i)),
        ],
        out_specs=pl.BlockSpec((rows_per_blk, 128), lambda i: (i, 0)),
        out_shape=jax.ShapeDtypeStruct((g * rows_per_blk, 128), jnp.float32),
    )(wcol, tableT)


def _sc_gather(proj1d, idx_by_worker, *, num_workers, n_chunks, chunk):
    bpw = n_chunks * chunk
    mesh = plsc.VectorSubcoreMesh(core_axis_name="c", subcore_axis_name="s")

    @functools.partial(
        pl.kernel,
        mesh=mesh,
        out_type=jax.ShapeDtypeStruct((num_workers * bpw,), jnp.float32),
        scratch_types=[
            pltpu.VMEM((n_chunks, chunk), jnp.int32),
            pltpu.VMEM((bpw,), jnp.float32),
            pltpu.SemaphoreType.DMA,
        ],
    )
    def k(proj_hbm, idx_hbm, out_hbm, idx_v, md_v, sem):
        wid = lax.axis_index("s") * 2 + lax.axis_index("c")
        base = wid * bpw
        pltpu.sync_copy(idx_hbm.at[wid], idx_v)

        def body(j, carry):
            off = pl.multiple_of(j * chunk, 8)
            pltpu.make_async_copy(
                proj_hbm.at[idx_v.at[j]], md_v.at[pl.ds(off, chunk)], sem
            ).start()
            return carry

        lax.fori_loop(0, n_chunks, body, 0)
        pltpu.make_async_copy(
            out_hbm.at[pl.ds(base, bpw)], md_v, sem
        ).wait()
        pltpu.sync_copy(md_v, out_hbm.at[pl.ds(base, bpw)])

    return k(proj1d, idx_by_worker)


_CHEB_M = 32
import numpy as _np

_j = _np.arange(_CHEB_M)
_theta = _np.pi * (_j + 0.5) / _CHEB_M
_CHEB_T = ((_np.cos(_theta) + 1.0) / 2.0).astype(_np.float32)
_D = _np.cos(_np.outer(_np.arange(_CHEB_M), _theta)) * (2.0 / _CHEB_M)
_D[0] *= 0.5
_CHEB_D = _D.astype(_np.float32)


def _time_coeffs(time_w, time_b, wt, rb):
    tj = jnp.asarray(_CHEB_T)
    f = rb[0] + jnp.sum(
        wt[None, :] * jnp.cos(tj[:, None] * time_w[None, :] + time_b[None, :]),
        axis=1,
    )
    return jnp.asarray(_CHEB_D) @ f


def _tc_finish(md2d, t2d, coeffs, *, block_rows):
    n_rows = t2d.shape[0]
    grid = n_rows // block_rows

    def body(c_s, md_ref, t_ref, o_ref):
        u2 = 4.0 * t_ref[...] - 2.0
        b1 = jnp.zeros_like(u2)
        b2 = jnp.zeros_like(u2)
        for n in range(_CHEB_M - 1, 0, -1):
            b1, b2 = c_s[n] + u2 * b1 - b2, b1
        tpart = c_s[0] + 0.5 * u2 * b1 - b2
        o_ref[...] = jax.nn.sigmoid(md_ref[...] + tpart)

    return pl.pallas_call(
        body,
        grid=(grid,),
        in_specs=[
            pl.BlockSpec(memory_space=pltpu.SMEM),
            pl.BlockSpec((block_rows, 128), lambda g: (g, 0)),
            pl.BlockSpec((block_rows, 128), lambda g: (g, 0)),
        ],
        out_specs=pl.BlockSpec((block_rows, 128), lambda g: (g, 0)),
        out_shape=jax.ShapeDtypeStruct((n_rows, 128), jnp.float32),
    )(coeffs, md2d, t2d)


def kernel(node_ids, times, memory_states, time_w, time_b, restart_W, restart_b):
    B, W = node_ids.shape
    bw = B * W
    num_workers = 32
    chunk = 128
    bpw = bw // num_workers
    n_chunks = bpw // chunk

    idx = node_ids.T.astype(jnp.int32).reshape(-1)
    idx_by_worker = idx.reshape(num_workers, n_chunks, chunk)

    wmem = restart_W[:MEM_DIM, 0]
    tableT = memory_states.T
    proj2d = _tc_proj(tableT, wmem.reshape(MEM_DIM, 1), cb=65536)
    proj1d = proj2d.reshape(-1)

    md = _sc_gather(proj1d, idx_by_worker,
                    num_workers=num_workers, n_chunks=n_chunks, chunk=chunk)

    t2d = times.T.reshape(bw // 128, 128)
    wt = restart_W[MEM_DIM:, 0]
    rb = restart_b.reshape((1,))
    coeffs = _time_coeffs(time_w, time_b, wt, rb)
    md2d = md.reshape(bw // 128, 128)
    probs = _tc_finish(md2d, t2d, coeffs, block_rows=256)
    return probs.reshape(W, B).T


# speedup vs baseline: 1.0066x; 1.0066x over previous
import functools

import jax
import jax.numpy as jnp
from jax import lax
from jax.experimental import pallas as pl
from jax.experimental.pallas import tpu as pltpu
from jax.experimental.pallas import tpu_sc as plsc

MEM_DIM = 32
TIME_DIM = 64

def _tc_proj(tableT, wcol, *, cb):
    n = tableT.shape[1]
    g = pl.cdiv(n, cb)
    rows_per_blk = cb // 128

    def body(w_ref, t_ref, o_ref):
        s = jnp.sum(t_ref[...] * w_ref[...], axis=0)
        o_ref[...] = s.reshape(rows_per_blk, 128)

    return pl.pallas_call(
        body,
        grid=(g,),
        in_specs=[
            pl.BlockSpec((MEM_DIM, 1), lambda i: (0, 0)),
            pl.BlockSpec((MEM_DIM, cb), lambda i: (0, i)),
        ],
        out_specs=pl.BlockSpec((rows_per_blk, 128), lambda i: (i, 0)),
        out_shape=jax.ShapeDtypeStruct((g * rows_per_blk, 128), jnp.float32),
    )(wcol, tableT)


def _sc_gather(proj1d, idx_by_worker, *, num_workers, n_chunks, chunk):
    bpw = n_chunks * chunk
    mesh = plsc.VectorSubcoreMesh(core_axis_name="c", subcore_axis_name="s")

    @functools.partial(
        pl.kernel,
        mesh=mesh,
        out_type=jax.ShapeDtypeStruct((num_workers * bpw,), jnp.float32),
        scratch_types=[
            pltpu.VMEM((n_chunks, chunk), jnp.int32),
            pltpu.VMEM((bpw,), jnp.float32),
            pltpu.SemaphoreType.DMA,
        ],
    )
    def k(proj_hbm, idx_hbm, out_hbm, idx_v, md_v, sem):
        wid = lax.axis_index("s") * 2 + lax.axis_index("c")
        base = wid * bpw
        pltpu.sync_copy(idx_hbm.at[wid], idx_v)

        def body(j, carry):
            off = pl.multiple_of(j * chunk, 8)
            pltpu.make_async_copy(
                proj_hbm.at[idx_v.at[j]], md_v.at[pl.ds(off, chunk)], sem
            ).start()
            return carry

        lax.fori_loop(0, n_chunks, body, 0)
        pltpu.make_async_copy(
            out_hbm.at[pl.ds(base, bpw)], md_v, sem
        ).wait()
        pltpu.sync_copy(md_v, out_hbm.at[pl.ds(base, bpw)])

    return k(proj1d, idx_by_worker)


_CHEB_M = 32
import numpy as _np

_j = _np.arange(_CHEB_M)
_theta = _np.pi * (_j + 0.5) / _CHEB_M
_CHEB_T = ((_np.cos(_theta) + 1.0) / 2.0).astype(_np.float32)
_D = _np.cos(_np.outer(_np.arange(_CHEB_M), _theta)) * (2.0 / _CHEB_M)
_D[0] *= 0.5
_CHEB_D = _D.astype(_np.float32)


def _time_coeffs(time_w, time_b, wt, rb):
    tj = jnp.asarray(_CHEB_T)
    f = rb[0] + jnp.sum(
        wt[None, :] * jnp.cos(tj[:, None] * time_w[None, :] + time_b[None, :]),
        axis=1,
    )
    return jnp.asarray(_CHEB_D) @ f


def _tc_time(t2d, coeffs, *, block_rows):
    n_rows = t2d.shape[0]
    grid = n_rows // block_rows

    def body(c_s, t_ref, o_ref):
        u2 = 4.0 * t_ref[...] - 2.0
        b1 = jnp.zeros_like(u2)
        b2 = jnp.zeros_like(u2)
        for n in range(_CHEB_M - 1, 0, -1):
            b1, b2 = c_s[n] + u2 * b1 - b2, b1
        o_ref[...] = c_s[0] + 0.5 * u2 * b1 - b2

    return pl.pallas_call(
        body,
        grid=(grid,),
        in_specs=[
            pl.BlockSpec(memory_space=pltpu.SMEM),
            pl.BlockSpec((block_rows, 128), lambda g: (g, 0)),
        ],
        out_specs=pl.BlockSpec((block_rows, 128), lambda g: (g, 0)),
        out_shape=jax.ShapeDtypeStruct((n_rows, 128), jnp.float32),
    )(coeffs, t2d)


def _tc_combine(md2d, tpart2d, *, block_rows):
    n_rows = md2d.shape[0]
    grid = n_rows // block_rows

    def body(md_ref, tp_ref, o_ref):
        o_ref[...] = jax.nn.sigmoid(md_ref[...] + tp_ref[...])

    return pl.pallas_call(
        body,
        grid=(grid,),
        in_specs=[
            pl.BlockSpec((block_rows, 128), lambda g: (g, 0)),
            pl.BlockSpec((block_rows, 128), lambda g: (g, 0)),
        ],
        out_specs=pl.BlockSpec((block_rows, 128), lambda g: (g, 0)),
        out_shape=jax.ShapeDtypeStruct((n_rows, 128), jnp.float32),
    )(md2d, tpart2d)


def kernel(node_ids, times, memory_states, time_w, time_b, restart_W, restart_b):
    B, W = node_ids.shape
    bw = B * W
    num_workers = 32
    chunk = 128
    bpw = bw // num_workers
    n_chunks = bpw // chunk

    idx = node_ids.T.astype(jnp.int32).reshape(-1)
    idx_by_worker = idx.reshape(num_workers, n_chunks, chunk)

    wmem = restart_W[:MEM_DIM, 0]
    tableT = memory_states.T
    proj2d = _tc_proj(tableT, wmem.reshape(MEM_DIM, 1), cb=65536)
    proj1d = proj2d.reshape(-1)

    md = _sc_gather(proj1d, idx_by_worker,
                    num_workers=num_workers, n_chunks=n_chunks, chunk=chunk)

    t2d = times.T.reshape(bw // 128, 128)
    wt = restart_W[MEM_DIM:, 0]
    rb = restart_b.reshape((1,))
    coeffs = _time_coeffs(time_w, time_b, wt, rb)
    tpart2d = _tc_time(t2d, coeffs, block_rows=256)

    md2d = md.reshape(bw // 128, 128)
    probs = _tc_combine(md2d, tpart2d, block_rows=256)
    return probs.reshape(W, B).T


# speedup vs baseline: 1.0329x; 1.0261x over previous
import functools

import jax
import jax.numpy as jnp
from jax import lax
from jax.experimental import pallas as pl
from jax.experimental.pallas import tpu as pltpu
from jax.experimental.pallas import tpu_sc as plsc

MEM_DIM = 32
TIME_DIM = 64

def _tc_proj(tableT, wcol, *, cb):
    n = tableT.shape[1]
    g = pl.cdiv(n, cb)
    rows_per_blk = cb // 128

    def body(w_ref, t_ref, o_ref):
        s = jnp.sum(t_ref[...] * w_ref[...], axis=0)
        o_ref[...] = s.reshape(rows_per_blk, 128)

    return pl.pallas_call(
        body,
        grid=(g,),
        in_specs=[
            pl.BlockSpec((MEM_DIM, 1), lambda i: (0, 0)),
            pl.BlockSpec((MEM_DIM, cb), lambda i: (0, i)),
        ],
        out_specs=pl.BlockSpec((rows_per_blk, 128), lambda i: (i, 0)),
        out_shape=jax.ShapeDtypeStruct((g * rows_per_blk, 128), jnp.float32),
    )(wcol, tableT)


def _sc_gather(proj1d, idx_by_worker, *, num_workers, n_chunks, chunk):
    bpw = n_chunks * chunk
    mesh = plsc.VectorSubcoreMesh(core_axis_name="c", subcore_axis_name="s")

    @functools.partial(
        pl.kernel,
        mesh=mesh,
        out_type=jax.ShapeDtypeStruct((num_workers * bpw,), jnp.float32),
        scratch_types=[
            pltpu.VMEM((n_chunks, chunk), jnp.int32),
            pltpu.VMEM((bpw,), jnp.float32),
            pltpu.SemaphoreType.DMA,
        ],
    )
    def k(proj_hbm, idx_hbm, out_hbm, idx_v, md_v, sem):
        wid = lax.axis_index("s") * 2 + lax.axis_index("c")
        base = wid * bpw
        pltpu.sync_copy(idx_hbm.at[wid], idx_v)

        def body(j, carry):
            off = pl.multiple_of(j * chunk, 8)
            pltpu.make_async_copy(
                proj_hbm.at[idx_v.at[j]], md_v.at[pl.ds(off, chunk)], sem
            ).start()
            return carry

        lax.fori_loop(0, n_chunks, body, 0)
        pltpu.make_async_copy(
            out_hbm.at[pl.ds(base, bpw)], md_v, sem
        ).wait()
        pltpu.sync_copy(md_v, out_hbm.at[pl.ds(base, bpw)])

    return k(proj1d, idx_by_worker)


_CHEB_M = 32
import numpy as _np

_j = _np.arange(_CHEB_M)
_theta = _np.pi * (_j + 0.5) / _CHEB_M
_CHEB_T = ((_np.cos(_theta) + 1.0) / 2.0).astype(_np.float32)
_D = _np.cos(_np.outer(_np.arange(_CHEB_M), _theta)) * (2.0 / _CHEB_M)
_D[0] *= 0.5
_CHEB_D = _D.astype(_np.float32)


def _time_coeffs(time_w, time_b, wt, rb):
    tj = jnp.asarray(_CHEB_T)
    f = rb[0] + jnp.sum(
        wt[None, :] * jnp.cos(tj[:, None] * time_w[None, :] + time_b[None, :]),
        axis=1,
    )
    return jnp.asarray(_CHEB_D) @ f


def _tc_time(t2d, coeffs, *, block_rows):
    n_rows = t2d.shape[0]
    grid = n_rows // block_rows

    def body(c_s, t_ref, o_ref):
        u2 = 4.0 * t_ref[...] - 2.0
        b1 = jnp.zeros_like(u2)
        b2 = jnp.zeros_like(u2)
        for n in range(_CHEB_M - 1, 0, -1):
            b1, b2 = c_s[n] + u2 * b1 - b2, b1
        o_ref[...] = c_s[0] + 0.5 * u2 * b1 - b2

    return pl.pallas_call(
        body,
        grid=(grid,),
        in_specs=[
            pl.BlockSpec(memory_space=pltpu.SMEM),
            pl.BlockSpec((block_rows, 128), lambda g: (g, 0)),
        ],
        out_specs=pl.BlockSpec((block_rows, 128), lambda g: (g, 0)),
        out_shape=jax.ShapeDtypeStruct((n_rows, 128), jnp.float32),
    )(coeffs, t2d)


def _tc_combine(md2d, tpart2d, *, block_rows):
    n_rows = md2d.shape[0]
    grid = n_rows // block_rows

    def body(md_ref, tp_ref, o_ref):
        o_ref[...] = jax.nn.sigmoid(md_ref[...] + tp_ref[...])

    return pl.pallas_call(
        body,
        grid=(grid,),
        in_specs=[
            pl.BlockSpec((block_rows, 128), lambda g: (g, 0)),
            pl.BlockSpec((block_rows, 128), lambda g: (g, 0)),
        ],
        out_specs=pl.BlockSpec((block_rows, 128), lambda g: (g, 0)),
        out_shape=jax.ShapeDtypeStruct((n_rows, 128), jnp.float32),
    )(md2d, tpart2d)


def kernel(node_ids, times, memory_states, time_w, time_b, restart_W, restart_b):
    B, W = node_ids.shape
    bw = B * W
    num_workers = 32
    chunk = 128
    bpw = bw // num_workers
    n_chunks = bpw // chunk

    idx = node_ids.T.astype(jnp.int32).reshape(-1)
    idx_by_worker = idx.reshape(num_workers, n_chunks, chunk)

    wmem = restart_W[:MEM_DIM, 0]
    tableT = memory_states.T
    proj2d = _tc_proj(tableT, wmem.reshape(MEM_DIM, 1), cb=131072)
    proj1d = proj2d.reshape(-1)

    md = _sc_gather(proj1d, idx_by_worker,
                    num_workers=num_workers, n_chunks=n_chunks, chunk=chunk)

    t2d = times.T.reshape(bw // 128, 128)
    wt = restart_W[MEM_DIM:, 0]
    rb = restart_b.reshape((1,))
    coeffs = _time_coeffs(time_w, time_b, wt, rb)
    tpart2d = _tc_time(t2d, coeffs, block_rows=256)

    md2d = md.reshape(bw // 128, 128)
    probs = _tc_combine(md2d, tpart2d, block_rows=1280)
    return probs.reshape(W, B).T
